# baseline (speedup 1.0000x reference)
import functools

import jax
import jax.numpy as jnp
from jax import lax
from jax.experimental import pallas as pl
from jax.experimental.pallas import tpu as pltpu
from jax.experimental.pallas import tpu_sc as plsc

B = 16384
DIM_CONTEXT = 128
EMB_DIM = 32
PACK = 128 // EMB_DIM

NC = 2
NS = 16
NW = NC * NS
ROWS_PER_W = B // NW
CHUNK = 128
NCHUNK = ROWS_PER_W // CHUNK

_sc_mesh = plsc.VectorSubcoreMesh(core_axis_name="c", subcore_axis_name="s")


@functools.partial(
    pl.kernel,
    mesh=_sc_mesh,
    out_type=jax.ShapeDtypeStruct((B, 128), jnp.float32),
    scratch_types=[
        pltpu.VMEM((NCHUNK, CHUNK), jnp.int32),
        pltpu.VMEM((ROWS_PER_W, 128), jnp.float32),
        pltpu.SemaphoreType.DMA,
    ],
)
def _sc_gather(table_hbm, idx_hbm, out_hbm, idx_v, rows_v, sem):
    wid = lax.axis_index("s") * NC + lax.axis_index("c")
    base = wid * ROWS_PER_W
    pltpu.sync_copy(idx_hbm.at[wid], idx_v)
    copies = []
    for j in range(NCHUNK):
        copies.append(
            pltpu.async_copy(
                table_hbm.at[idx_v.at[j]],
                rows_v.at[pl.ds(j * CHUNK, CHUNK)],
                sem,
            )
        )
    for c in copies:
        c.wait()
    pltpu.sync_copy(rows_v, out_hbm.at[pl.ds(base, ROWS_PER_W)])


_GRID = 16
_RB = B // _GRID


def _tc_body(ctx_ref, w_ref, b_ref, off_ref, emb4_ref, out_ref):
    proj = lax.dot_general(
        ctx_ref[...], w_ref[...], (((1,), (1,)), ((), ())),
        preferred_element_type=jnp.float32,
    ) + b_ref[...]
    projt = jnp.concatenate([proj, proj, proj, proj], axis=1)
    lane = lax.broadcasted_iota(jnp.int32, (_RB, 128), 1)
    off = off_ref[...]
    sel = (lane >= off) & (lane < off + EMB_DIM)
    out_ref[...] = jnp.sum(
        jnp.where(sel, projt * emb4_ref[...], 0.0), axis=1
    )


_tc_combine = pl.pallas_call(
    _tc_body,
    grid=(_GRID,),
    in_specs=[
        pl.BlockSpec((_RB, DIM_CONTEXT), lambda i: (i, 0)),
        pl.BlockSpec((EMB_DIM, DIM_CONTEXT), lambda i: (0, 0)),
        pl.BlockSpec((1, EMB_DIM), lambda i: (0, 0)),
        pl.BlockSpec((_RB, 1), lambda i: (i, 0)),
        pl.BlockSpec((_RB, 128), lambda i: (i, 0)),
    ],
    out_specs=pl.BlockSpec((_RB,), lambda i: (i,)),
    out_shape=jax.ShapeDtypeStruct((B,), jnp.float32),
)


def kernel(context, item_ids, W, b, table):
    ids = item_ids.astype(jnp.int32)
    ids4 = (ids // PACK).reshape(NW, NCHUNK, CHUNK)
    offs = ((ids % PACK) * EMB_DIM).reshape(B, 1)
    table4 = table.reshape(table.shape[0] // PACK, 128)
    emb4 = _sc_gather(table4, ids4)
    return _tc_combine(context, W, b.reshape(1, EMB_DIM), offs, emb4)


# speedup vs baseline: 1.0208x; 1.0208x over previous
import functools

import jax
import jax.numpy as jnp
from jax import lax
from jax.experimental import pallas as pl
from jax.experimental.pallas import tpu as pltpu
from jax.experimental.pallas import tpu_sc as plsc

B = 16384
DIM_CONTEXT = 128
EMB_DIM = 32

NC = 2
NS = 16
NW = NC * NS
ROWS_PER_W = B // NW
CHUNK = 128
NCHUNK = ROWS_PER_W // CHUNK

_sc_mesh = plsc.VectorSubcoreMesh(core_axis_name="c", subcore_axis_name="s")


@functools.partial(
    pl.kernel,
    mesh=_sc_mesh,
    out_type=jax.ShapeDtypeStruct((B, EMB_DIM), jnp.float32),
    scratch_types=[
        pltpu.VMEM((NCHUNK, CHUNK), jnp.int32),
        pltpu.VMEM((ROWS_PER_W, EMB_DIM), jnp.float32),
        pltpu.SemaphoreType.DMA,
    ],
    compiler_params=pltpu.CompilerParams(use_tc_tiling_on_sc=False),
)
def _sc_gather(table_hbm, idx_hbm, out_hbm, idx_v, rows_v, sem):
    wid = lax.axis_index("s") * NC + lax.axis_index("c")
    base = wid * ROWS_PER_W
    pltpu.sync_copy(idx_hbm.at[wid], idx_v)
    copies = []
    for j in range(NCHUNK):
        copies.append(
            pltpu.async_copy(
                table_hbm.at[idx_v.at[j]],
                rows_v.at[pl.ds(j * CHUNK, CHUNK)],
                sem,
            )
        )
    for c in copies:
        c.wait()
    pltpu.sync_copy(rows_v, out_hbm.at[pl.ds(base, ROWS_PER_W)])


_GRID = 16
_RB = B // _GRID


def _tc_body(ctx_ref, w_ref, b_ref, emb_ref, out_ref):
    proj = lax.dot_general(
        ctx_ref[...], w_ref[...], (((1,), (1,)), ((), ())),
        preferred_element_type=jnp.float32,
    ) + b_ref[...]
    out_ref[...] = jnp.sum(proj * emb_ref[...], axis=1)


_tc_combine = pl.pallas_call(
    _tc_body,
    grid=(_GRID,),
    in_specs=[
        pl.BlockSpec((_RB, DIM_CONTEXT), lambda i: (i, 0)),
        pl.BlockSpec((EMB_DIM, DIM_CONTEXT), lambda i: (0, 0)),
        pl.BlockSpec((1, EMB_DIM), lambda i: (0, 0)),
        pl.BlockSpec((_RB, EMB_DIM), lambda i: (i, 0)),
    ],
    out_specs=pl.BlockSpec((_RB,), lambda i: (i,)),
    out_shape=jax.ShapeDtypeStruct((B,), jnp.float32),
)


def kernel(context, item_ids, W, b, table):
    ids = item_ids.astype(jnp.int32).reshape(NW, NCHUNK, CHUNK)
    emb = _sc_gather(table, ids)
    return _tc_combine(context, W, b.reshape(1, EMB_DIM), emb)


# speedup vs baseline: 3.8633x; 3.7847x over previous
import functools

import jax
import jax.numpy as jnp
from jax import lax
from jax.experimental import pallas as pl
from jax.experimental.pallas import tpu as pltpu
from jax.experimental.pallas import tpu_sc as plsc

B = 16384
DIM_CONTEXT = 128
EMB_DIM = 32
N_ITEMS = 1000000

NC = 2
NS = 16
NW = NC * NS
ROWS_PER_W = B // NW
CHUNK = 128
NCHUNK = ROWS_PER_W // CHUNK
GROUP = 16
NGROUP = ROWS_PER_W // GROUP
LANES = 128

_sc_mesh = plsc.VectorSubcoreMesh(core_axis_name="c", subcore_axis_name="s")


@functools.partial(
    pl.kernel,
    mesh=_sc_mesh,
    out_type=jax.ShapeDtypeStruct((B,), jnp.float32),
    scratch_types=[
        pltpu.VMEM((NCHUNK, CHUNK), jnp.int32),
        pltpu.VMEM((GROUP * EMB_DIM, LANES), jnp.float32),
        pltpu.VMEM((EMB_DIM * ROWS_PER_W,), jnp.float32),
        pltpu.VMEM((ROWS_PER_W,), jnp.float32),
        pltpu.SemaphoreType.DMA,
        pltpu.SemaphoreType.DMA,
    ],
    compiler_params=pltpu.CompilerParams(
        disable_bounds_checks=True, needs_layout_passes=False),
)
def _sc_gather_dot(tabT_hbm, ids_hbm, projT_hbm, out_hbm,
                   idx_v, blk_v, pj_v, out_v, sem, psem):
    wid = lax.axis_index("s") * NC + lax.axis_index("c")
    base = wid * ROWS_PER_W
    pltpu.sync_copy(ids_hbm.at[wid], idx_v)
    pj_copies = [
        pltpu.async_copy(
            projT_hbm.at[d, pl.ds(base, ROWS_PER_W)],
            pj_v.at[pl.ds(d * ROWS_PER_W, ROWS_PER_W)],
            psem,
        )
        for d in range(EMB_DIM)
    ]
    for c in pj_copies:
        c.wait()

    rows_base = lax.iota(jnp.int32, GROUP) * EMB_DIM

    def group_body(g, carry):
        chunk = g // (CHUNK // GROUP)
        lane0 = (g % (CHUNK // GROUP)) * GROUP
        cols = idx_v[chunk, pl.ds(lane0, GROUP)]
        off = lax.rem(cols, jnp.int32(LANES))
        copies = []
        for j in range(GROUP):
            col_al = pl.multiple_of(
                (cols[j] // LANES) * LANES, LANES)
            copies.append(pltpu.make_async_copy(
                tabT_hbm.at[:, pl.ds(col_al, LANES)],
                blk_v.at[pl.ds(j * EMB_DIM, EMB_DIM), :],
                sem,
            ))
        for c in copies:
            c.start()
        for c in copies:
            c.wait()

        k0 = g * GROUP
        acc = jnp.zeros((GROUP,), jnp.float32)
        for d in range(EMB_DIM):
            v = plsc.load_gather(blk_v, [rows_base + d, off])
            acc = acc + v * pj_v[pl.ds(d * ROWS_PER_W + k0, GROUP)]
        out_v[pl.ds(k0, GROUP)] = acc
        return carry

    lax.fori_loop(0, NGROUP, group_body, 0)
    pltpu.sync_copy(out_v, out_hbm.at[pl.ds(base, ROWS_PER_W)])


_PBLK = 2048
_PGRID = B // _PBLK


def _tc_proj_body(w_ref, ctx_ref, bb_ref, out_ref):
    proj = lax.dot_general(
        w_ref[...], ctx_ref[...], (((1,), (1,)), ((), ())),
        preferred_element_type=jnp.float32,
    )
    bias = jnp.broadcast_to(bb_ref[...][:, 0:1], (EMB_DIM, _PBLK))
    out_ref[...] = proj + bias


_tc_proj = pl.pallas_call(
    _tc_proj_body,
    grid=(_PGRID,),
    in_specs=[
        pl.BlockSpec((EMB_DIM, DIM_CONTEXT), lambda i: (0, 0)),
        pl.BlockSpec((_PBLK, DIM_CONTEXT), lambda i: (i, 0)),
        pl.BlockSpec((EMB_DIM, 128), lambda i: (0, 0)),
    ],
    out_specs=pl.BlockSpec((EMB_DIM, _PBLK), lambda i: (0, i)),
    out_shape=jax.ShapeDtypeStruct((EMB_DIM, B), jnp.float32),
)


def kernel(context, item_ids, W, b, table):
    ids = item_ids.astype(jnp.int32).reshape(NW, NCHUNK, CHUNK)
    bb = jnp.broadcast_to(b.reshape(EMB_DIM, 1), (EMB_DIM, 128))
    projT = _tc_proj(W, context, bb)
    return _sc_gather_dot(table.T, ids, projT)
